# baseline (speedup 1.0000x reference)
import functools

import jax
import jax.numpy as jnp
from jax import lax
from jax.experimental import pallas as pl
from jax.experimental.pallas import tpu as pltpu, tpu_sc as plsc

N = 4096
C = 64
NBINS = 16
L = 16
JQ = N // 4
NSC = 512
G = NSC // 8
RC = 16
NR = G // RC
BI = 128
GRID_F = RC * NBINS * C
DUMP = GRID_F
UNROLL = 8

_GDN = lax.GatherDimensionNumbers(offset_dims=(), collapsed_slice_dims=(0,),
                                  start_index_map=(0,))


def _lane_bcast(vec, kk):
    idx = jnp.full((L, 1), kk, jnp.int32)
    return lax.gather(vec, idx, dimension_numbers=_GDN, slice_sizes=(1,),
                      mode=lax.GatherScatterMode.PROMISE_IN_BOUNDS)


def _sc_body(attr_hbm, xh, yh, zh, zeros_hbm, out_hbm,
             xc, yc, zc, attrq, jl, dl, grids, spm):
    cid = lax.axis_index("c")
    sid = lax.axis_index("s")
    h = sid // 4
    q = sid % 4
    pltpu.sync_copy(xh, xc)
    pltpu.sync_copy(yh, yc)
    pltpu.sync_copy(zh, zc)
    pltpu.sync_copy(attr_hbm.at[pl.ds(q * JQ * C, JQ * C)], attrq)
    iota = lax.iota(jnp.int32, L)
    iota64 = iota * C
    c16 = [iota + L * t for t in range(4)]
    zvec = jnp.zeros((L,), jnp.int32)
    jq0 = q * JQ
    c00 = cid * (NSC // 2) + h * G

    def round_body(r, _):
        pltpu.sync_copy(zeros_hbm, grids)
        cr0 = c00 + r * RC

        def center_body(k, _):
            i = cr0 + k
            ivec = jnp.broadcast_to(i, (L,)).astype(jnp.int32)
            xi = plsc.load_gather(xc, [ivec])
            yi = plsc.load_gather(yc, [ivec])
            zi = plsc.load_gather(zc, [ivec])
            dbase = k * (NBINS * C)

            def group_body(g0, countm1):
                for u in range(UNROLL):
                    g = g0 * UNROLL + u
                    sl = pl.ds(jq0 + g * L, L)
                    dx = xc[sl] - xi
                    dy = yc[sl] - yi
                    dz = zc[sl] - zi
                    d2 = dx * dx + dy * dy + dz * dz
                    jv = jq0 + g * L + iota
                    valid = (d2 <= 1.0) & (jv != ivec)
                    dest = (jnp.where(d2 >= 0.25, dbase + 8 * C, dbase)
                            + jnp.where(dx > 0, 4 * C, 0)
                            + jnp.where(dy > 0, 2 * C, 0)
                            + jnp.where(dz > 0, C, 0))
                    jst = g * (L * C) + iota64
                    pos = countm1 + plsc.cumsum(valid.astype(jnp.int32))
                    plsc.store_scatter(jl, [pos], jst, mask=valid)
                    plsc.store_scatter(dl, [pos], dest, mask=valid)
                    countm1 = countm1 + plsc.all_reduce_population_count(valid)
                return countm1

            countm1 = lax.fori_loop(0, JQ // (L * UNROLL), group_body,
                                    zvec - 1)
            count = countm1 + 1
            dumpv = jnp.broadcast_to(DUMP, (L,)).astype(jnp.int32)
            plsc.store_scatter(jl, [count + iota], zvec)
            plsc.store_scatter(dl, [count + iota], dumpv)
            cnt = jnp.max(count)
            np16 = (cnt + L - 1) // L

            def pair_body(p, _):
                jvec = jl[pl.ds(p * L, L)]
                destv = dl[pl.ds(p * L, L)]
                for kk in range(L):
                    jb = _lane_bcast(jvec, kk)
                    db = _lane_bcast(destv, kk)
                    for t in range(4):
                        v = plsc.load_gather(attrq, [jb + c16[t]])
                        plsc.addupdate_scatter(grids, [db + c16[t]], v)
                return 0

            lax.fori_loop(0, np16, pair_body, 0)
            return 0

        lax.fori_loop(0, RC, center_body, 0)

        pltpu.sync_copy(grids.at[pl.ds(0, GRID_F)], spm.at[sid])
        plsc.subcore_barrier()
        slice_f = (RC // 4) * NBINS * C
        for p in range(4):
            pltpu.sync_copy(spm.at[h * 4 + p, pl.ds(q * slice_f, slice_f)],
                            grids.at[pl.ds(p * slice_f, slice_f)])

        def red_body(t, _):
            sl0 = pl.ds(t * L, L)
            acc = (grids[sl0]
                   + grids[pl.ds(slice_f + t * L, L)]
                   + grids[pl.ds(2 * slice_f + t * L, L)]
                   + grids[pl.ds(3 * slice_f + t * L, L)])
            grids[sl0] = acc
            return 0

        lax.fori_loop(0, slice_f // L, red_body, 0)
        out0 = (cr0 + q * (RC // 4)) * (NBINS * C)
        pltpu.sync_copy(grids.at[pl.ds(0, slice_f)],
                        out_hbm.at[pl.ds(out0, slice_f)])
        plsc.subcore_barrier()
        return 0

    lax.fori_loop(0, NR, round_body, 0)


def _sc_point_conv(in_attributes, point_cloud):
    xs = point_cloud[:, 0].reshape(N)
    ys = point_cloud[:, 1].reshape(N)
    zs = point_cloud[:, 2].reshape(N)
    attr_flat = in_attributes.reshape(N * C)
    zeros = jnp.zeros((GRID_F + C,), jnp.float32)
    mesh = plsc.VectorSubcoreMesh(core_axis_name="c", subcore_axis_name="s")
    k = functools.partial(
        pl.kernel,
        out_type=jax.ShapeDtypeStruct((NSC * NBINS * C,), jnp.float32),
        mesh=mesh,
        compiler_params=pltpu.CompilerParams(needs_layout_passes=False,
                                             use_tc_tiling_on_sc=False),
        scratch_types=[
            pltpu.VMEM((N,), jnp.float32),
            pltpu.VMEM((N,), jnp.float32),
            pltpu.VMEM((N,), jnp.float32),
            pltpu.VMEM((JQ * C,), jnp.float32),
            pltpu.VMEM((JQ + L,), jnp.int32),
            pltpu.VMEM((JQ + L,), jnp.int32),
            pltpu.VMEM((GRID_F + C,), jnp.float32),
            pltpu.VMEM_SHARED((16, GRID_F), jnp.float32),
        ],
    )(_sc_body)
    out = k(attr_flat, xs, ys, zs, zeros)
    return out.reshape(NSC, NBINS, C)


def _tc_body(xyz_ref, pci_ref, attr_ref, out_ref):
    i0 = NSC + pl.program_id(0) * BI
    xyz = xyz_ref[...]
    xj = xyz[0, :][None, :]
    yj = xyz[1, :][None, :]
    zj = xyz[2, :][None, :]
    pci = pci_ref[...]
    dx = xj - pci[:, 0:1]
    dy = yj - pci[:, 1:2]
    dz = zj - pci[:, 2:3]
    d2 = dx * dx + dy * dy + dz * dz + 1e-12
    dist = jnp.sqrt(d2)
    shell = jnp.where(dist >= 0.5, 8.0, 0.0)
    octant = (jnp.where(dx > 0, 4.0, 0.0)
              + jnp.where(dy > 0, 2.0, 0.0)
              + jnp.where(dz > 0, 1.0, 0.0))
    jidx = lax.broadcasted_iota(jnp.int32, (BI, N), 1)
    iidx = lax.broadcasted_iota(jnp.int32, (BI, N), 0) + i0
    valid = (dist <= 1.0) & (jidx != iidx)
    bin_eff = jnp.where(valid, shell + octant,
                        float(NBINS)).astype(jnp.bfloat16)
    attr = attr_ref[...]
    for b in range(NBINS):
        mask = (bin_eff == jnp.bfloat16(b)).astype(jnp.bfloat16)
        out_ref[:, b, :] = jnp.dot(mask, attr,
                                   preferred_element_type=jnp.float32)


def _tc_point_conv(in_attributes, point_cloud):
    xyz = point_cloud.T.reshape(3, N)
    pci = jnp.pad(point_cloud[NSC:], ((0, 0), (0, 5)))
    grid = ((N - NSC) // BI,)
    return pl.pallas_call(
        _tc_body,
        grid=grid,
        in_specs=[
            pl.BlockSpec((3, N), lambda i: (0, 0)),
            pl.BlockSpec((BI, 8), lambda i: (i, 0)),
            pl.BlockSpec((N, C), lambda i: (0, 0)),
        ],
        out_specs=pl.BlockSpec((BI, NBINS, C), lambda i: (i, 0, 0)),
        out_shape=jax.ShapeDtypeStruct((N - NSC, NBINS, C), jnp.float32),
    )(xyz, pci, in_attributes.astype(jnp.bfloat16))


@jax.jit
def kernel(in_attributes, point_cloud):
    out_sc = _sc_point_conv(in_attributes, point_cloud)
    out_tc = _tc_point_conv(in_attributes, point_cloud)
    return jnp.concatenate([out_sc, out_tc], axis=0)


# speedup vs baseline: 1.2880x; 1.2880x over previous
import functools

import jax
import jax.numpy as jnp
from jax import lax
from jax.experimental import pallas as pl
from jax.experimental.pallas import tpu as pltpu, tpu_sc as plsc

N = 4096
C = 64
NBINS = 16
L = 16
JQ = N // 4
NSC = 512
G = NSC // 8
RC = 16
NR = G // RC
BI = 128
GRID_F = RC * NBINS * C
DUMP = GRID_F
UNROLL = 8

_GDN = lax.GatherDimensionNumbers(offset_dims=(), collapsed_slice_dims=(0,),
                                  start_index_map=(0,))


def _lane_bcast(vec, kk):
    idx = jnp.full((L, 1), kk, jnp.int32)
    return lax.gather(vec, idx, dimension_numbers=_GDN, slice_sizes=(1,),
                      mode=lax.GatherScatterMode.PROMISE_IN_BOUNDS)


def _sc_body(attr_hbm, xh, yh, zh, zeros_hbm, out_hbm,
             xc, yc, zc, attrq, jl, dl, grids, spm):
    cid = lax.axis_index("c")
    sid = lax.axis_index("s")
    h = sid // 4
    q = sid % 4
    pltpu.sync_copy(xh, xc)
    pltpu.sync_copy(yh, yc)
    pltpu.sync_copy(zh, zc)
    pltpu.sync_copy(attr_hbm.at[pl.ds(q * JQ * C, JQ * C)], attrq)
    iota = lax.iota(jnp.int32, L)
    iota64 = iota * C
    c16 = [iota + L * t for t in range(4)]
    zvec = jnp.zeros((L,), jnp.int32)
    jq0 = q * JQ
    c00 = cid * (NSC // 2) + h * G

    def round_body(r, _):
        pltpu.sync_copy(zeros_hbm, grids)
        cr0 = c00 + r * RC

        def center_body(k, _):
            i = cr0 + k
            ivec = jnp.broadcast_to(i, (L,)).astype(jnp.int32)
            xi = plsc.load_gather(xc, [ivec])
            yi = plsc.load_gather(yc, [ivec])
            zi = plsc.load_gather(zc, [ivec])
            dbase = k * (NBINS * C)

            def group_body(g0, countm1):
                for u in range(UNROLL):
                    g = g0 * UNROLL + u
                    sl = pl.ds(jq0 + g * L, L)
                    dx = xc[sl] - xi
                    dy = yc[sl] - yi
                    dz = zc[sl] - zi
                    d2 = dx * dx + dy * dy + dz * dz
                    jv = jq0 + g * L + iota
                    valid = (d2 <= 1.0) & (jv != ivec)
                    dest = (jnp.where(d2 >= 0.25, dbase + 8 * C, dbase)
                            + jnp.where(dx > 0, 4 * C, 0)
                            + jnp.where(dy > 0, 2 * C, 0)
                            + jnp.where(dz > 0, C, 0))
                    jst = g * (L * C) + iota64
                    pos = countm1 + plsc.cumsum(valid.astype(jnp.int32))
                    plsc.store_scatter(jl, [pos], jst, mask=valid)
                    plsc.store_scatter(dl, [pos], dest, mask=valid)
                    countm1 = countm1 + plsc.all_reduce_population_count(valid)
                return countm1

            countm1 = lax.fori_loop(0, JQ // (L * UNROLL), group_body,
                                    zvec - 1)
            count = countm1 + 1
            dumpv = jnp.broadcast_to(DUMP, (L,)).astype(jnp.int32)
            plsc.store_scatter(jl, [count + iota], zvec)
            plsc.store_scatter(dl, [count + iota], dumpv)
            cnt = jnp.max(count)
            np16 = (cnt + L - 1) // L

            def pair_body(p, _):
                jvec = jl[pl.ds(p * L, L)]
                destv = dl[pl.ds(p * L, L)]
                for kk in range(L):
                    jb = _lane_bcast(jvec, kk)
                    db = _lane_bcast(destv, kk)
                    for t in range(4):
                        v = plsc.load_gather(attrq, [jb + c16[t]])
                        plsc.addupdate_scatter(grids, [db + c16[t]], v)
                return 0

            lax.fori_loop(0, np16, pair_body, 0)
            return 0

        lax.fori_loop(0, RC, center_body, 0)

        pltpu.sync_copy(grids.at[pl.ds(0, GRID_F)], spm.at[sid])
        plsc.subcore_barrier()
        slice_f = (RC // 4) * NBINS * C
        for p in range(4):
            pltpu.sync_copy(spm.at[h * 4 + p, pl.ds(q * slice_f, slice_f)],
                            grids.at[pl.ds(p * slice_f, slice_f)])

        def red_body(t, _):
            sl0 = pl.ds(t * L, L)
            acc = (grids[sl0]
                   + grids[pl.ds(slice_f + t * L, L)]
                   + grids[pl.ds(2 * slice_f + t * L, L)]
                   + grids[pl.ds(3 * slice_f + t * L, L)])
            grids[sl0] = acc
            return 0

        lax.fori_loop(0, slice_f // L, red_body, 0)
        out0 = (cr0 + q * (RC // 4)) * (NBINS * C)
        pltpu.sync_copy(grids.at[pl.ds(0, slice_f)],
                        out_hbm.at[pl.ds(out0, slice_f)])
        plsc.subcore_barrier()
        return 0

    lax.fori_loop(0, NR, round_body, 0)


def _sc_point_conv(in_attributes, point_cloud):
    xs = point_cloud[:, 0].reshape(N)
    ys = point_cloud[:, 1].reshape(N)
    zs = point_cloud[:, 2].reshape(N)
    attr_flat = in_attributes.reshape(N * C)
    zeros = jnp.zeros((GRID_F + C,), jnp.float32)
    mesh = plsc.VectorSubcoreMesh(core_axis_name="c", subcore_axis_name="s")
    k = functools.partial(
        pl.kernel,
        out_type=jax.ShapeDtypeStruct((NSC * NBINS * C,), jnp.float32),
        mesh=mesh,
        compiler_params=pltpu.CompilerParams(needs_layout_passes=False,
                                             use_tc_tiling_on_sc=False),
        scratch_types=[
            pltpu.VMEM((N,), jnp.float32),
            pltpu.VMEM((N,), jnp.float32),
            pltpu.VMEM((N,), jnp.float32),
            pltpu.VMEM((JQ * C,), jnp.float32),
            pltpu.VMEM((JQ + L,), jnp.int32),
            pltpu.VMEM((JQ + L,), jnp.int32),
            pltpu.VMEM((GRID_F + C,), jnp.float32),
            pltpu.VMEM_SHARED((16, GRID_F), jnp.float32),
        ],
    )(_sc_body)
    out = k(attr_flat, xs, ys, zs, zeros)
    return out.reshape(NSC, NBINS, C)


def _tc_body(xyz_ref, pci_ref, attr_ref, out_ref):
    i0 = NSC + pl.program_id(0) * BI
    xyz = xyz_ref[...]
    xj = xyz[0, :][None, :]
    yj = xyz[1, :][None, :]
    zj = xyz[2, :][None, :]
    pci = pci_ref[...]
    dx = xj - pci[:, 0:1]
    dy = yj - pci[:, 1:2]
    dz = zj - pci[:, 2:3]
    d2 = dx * dx + dy * dy + dz * dz + 1e-12
    dist = jnp.sqrt(d2)
    shell = (dist >= 0.5).astype(jnp.int32)
    octant = (4 * (dx > 0).astype(jnp.int32)
              + 2 * (dy > 0).astype(jnp.int32)
              + (dz > 0).astype(jnp.int32))
    bin_idx = shell * 8 + octant
    jidx = lax.broadcasted_iota(jnp.int32, (BI, N), 1)
    iidx = lax.broadcasted_iota(jnp.int32, (BI, N), 0) + i0
    valid = (dist <= 1.0) & (jidx != iidx)
    bin_eff = jnp.where(valid, bin_idx, NBINS)
    attr = attr_ref[...]
    for b in range(NBINS):
        mask = (bin_eff == b).astype(jnp.float32)
        out_ref[:, b, :] = jnp.dot(mask, attr,
                                   preferred_element_type=jnp.float32)


def _tc_point_conv(in_attributes, point_cloud):
    xyz = point_cloud.T.reshape(3, N)
    pci = jnp.pad(point_cloud[NSC:], ((0, 0), (0, 5)))
    grid = ((N - NSC) // BI,)
    return pl.pallas_call(
        _tc_body,
        grid=grid,
        in_specs=[
            pl.BlockSpec((3, N), lambda i: (0, 0)),
            pl.BlockSpec((BI, 8), lambda i: (i, 0)),
            pl.BlockSpec((N, C), lambda i: (0, 0)),
        ],
        out_specs=pl.BlockSpec((BI, NBINS, C), lambda i: (i + NSC // BI, 0, 0)),
        out_shape=jax.ShapeDtypeStruct((N, NBINS, C), jnp.float32),
    )(xyz, pci, in_attributes)


@jax.jit
def kernel(in_attributes, point_cloud):
    out_sc = _sc_point_conv(in_attributes, point_cloud)
    out_tc = _tc_point_conv(in_attributes, point_cloud)
    return lax.dynamic_update_slice(out_tc, out_sc, (0, 0, 0))
